# baseline (speedup 1.0000x reference)
import jax
import jax.numpy as jnp
from jax.experimental import pallas as pl


def _posadd_kernel(x_ref, pos_ref, o_ref):
    pos = pos_ref[0:500, :]
    o_ref[...] = x_ref[...] + pos[None, :, :]


def kernel(x, pos_table):
    B, S, D = x.shape
    BB = 8
    return pl.pallas_call(
        _posadd_kernel,
        grid=(B // BB,),
        in_specs=[
            pl.BlockSpec((BB, S, D), lambda i: (i, 0, 0)),
            pl.BlockSpec(pos_table.shape, lambda i: (0, 0)),
        ],
        out_specs=pl.BlockSpec((BB, S, D), lambda i: (i, 0, 0)),
        out_shape=jax.ShapeDtypeStruct((B, S, D), x.dtype),
    )(x, pos_table)
